# baseline (speedup 1.0000x reference)
import functools
import jax
import jax.numpy as jnp
from jax import lax
from jax.experimental import pallas as pl
from jax.experimental.pallas import tpu as pltpu

B, S, H, I, K = 4, 2048, 768, 3072, 384
TB = 256


def _moc_body(x_ref, gw_ref, uw_ref, dw_ref, o_ref):
    xb = x_ref[...]
    gate = lax.dot_general(gw_ref[...], xb,
                           (((1,), (1,)), ((), ())),
                           preferred_element_type=jnp.float32)
    up = lax.dot_general(uw_ref[...], xb.astype(jnp.bfloat16),
                         (((1,), (1,)), ((), ())),
                         preferred_element_type=jnp.float32)

    def enc(f):
        b = lax.bitcast_convert_type(f, jnp.uint32)
        return jnp.where(b >> 31 == 1, ~b, b | jnp.uint32(0x80000000))

    ukey = enc(gate)

    def count_ge(cand):
        return jnp.sum((ukey >= cand).astype(jnp.int32), axis=0, keepdims=True)

    m1 = jnp.sum(gate, axis=0, keepdims=True) * (1.0 / I)
    m2 = jnp.sum(gate * gate, axis=0, keepdims=True) * (1.0 / I)
    sigma = jnp.sqrt(jnp.maximum(m2 - m1 * m1, 1e-30))
    t0 = m1 + 1.1503494 * sigma

    lo = jnp.zeros((1, TB), dtype=jnp.uint32)
    hi = jnp.full((1, TB), jnp.uint32(0xFFFFFFFF))
    thr = jnp.zeros((1, TB), dtype=jnp.uint32)
    done = jnp.zeros((1, TB), dtype=jnp.int32)

    def absorb(midu, cnt, lo, hi, thr, done):
        ge = cnt >= K
        nlo = jnp.where(ge, midu, lo)
        nhi = jnp.where(ge, hi, midu)
        closed = nlo + jnp.uint32(1) >= nhi
        hit = cnt == K
        nthr = jnp.where((done == 0) & (hit | closed),
                         jnp.where(hit, midu, nlo), thr)
        ndone = jnp.where(hit | closed, jnp.int32(1), done)
        return nlo, nhi, nthr, ndone

    c0 = count_ge(enc(t0))
    lo, hi, thr, done = absorb(enc(t0), c0, lo, hi, thr, done)
    t1 = t0 + (c0 - K).astype(jnp.float32) * sigma * (1.0 / (I * 0.20594))
    c1 = count_ge(enc(t1))
    lo, hi, thr, done = absorb(enc(t1), c1, lo, hi, thr, done)

    def cond(state):
        i, lo, hi, thr, done = state
        return jnp.logical_and(i < 34, jnp.any(done == 0))

    def body(state):
        i, lo, hi, thr, done = state
        midu = lo + ((hi - lo) >> 1)
        cnt = count_ge(midu)
        lo, hi, thr, done = absorb(midu, cnt, lo, hi, thr, done)
        return (i + 1, lo, hi, thr, done)

    _, lo, hi, thr, done = lax.while_loop(
        cond, body, (jnp.int32(0), lo, hi, thr, done))

    mask = ukey >= thr
    act = gate * jax.nn.sigmoid(gate) * up
    masked = jnp.where(mask, act, 0.0).astype(jnp.bfloat16)
    o_ref[...] = lax.dot_general(masked, dw_ref[...],
                                 (((0,), (1,)), ((), ())),
                                 preferred_element_type=jnp.float32)


@jax.jit
def kernel(x, gate_w, up_w, down_w):
    b, s, h = x.shape
    T = b * s
    x2 = x.reshape(T, h)
    up_w = up_w.astype(jnp.bfloat16)
    down_w = down_w.astype(jnp.bfloat16)
    out = pl.pallas_call(
        _moc_body,
        grid=(T // TB,),
        in_specs=[
            pl.BlockSpec((TB, H), lambda i: (i, 0)),
            pl.BlockSpec((I, H), lambda i: (0, 0)),
            pl.BlockSpec((I, H), lambda i: (0, 0)),
            pl.BlockSpec((H, I), lambda i: (0, 0)),
        ],
        out_specs=pl.BlockSpec((TB, H), lambda i: (i, 0)),
        out_shape=jax.ShapeDtypeStruct((T, H), jnp.float32),
    )(x2, gate_w, up_w, down_w)
    return out.reshape(b, s, h)


# speedup vs baseline: 1.2500x; 1.2500x over previous
import functools
import jax
import jax.numpy as jnp
from jax import lax
from jax.experimental import pallas as pl
from jax.experimental.pallas import tpu as pltpu

B, S, H, I, K = 4, 2048, 768, 3072, 384
TB = 256


def _moc_body(x_ref, gw_ref, uw_ref, dw_ref, o_ref):
    xb = x_ref[...]
    gate = lax.dot_general(gw_ref[...], xb,
                           (((1,), (1,)), ((), ())),
                           preferred_element_type=jnp.float32)
    up = lax.dot_general(uw_ref[...], xb.astype(jnp.bfloat16),
                         (((1,), (1,)), ((), ())),
                         preferred_element_type=jnp.float32)

    def enc(f):
        b = lax.bitcast_convert_type(f, jnp.uint32)
        return jnp.where(b >> 31 == 1, ~b, b | jnp.uint32(0x80000000))

    ukey = enc(gate)

    def count_ge(cand):
        return jnp.sum((ukey >= cand).astype(jnp.int32), axis=0, keepdims=True)

    m1 = jnp.sum(gate, axis=0, keepdims=True) * (1.0 / I)
    m2 = jnp.sum(gate * gate, axis=0, keepdims=True) * (1.0 / I)
    sigma = jnp.sqrt(jnp.maximum(m2 - m1 * m1, 1e-30))
    t0 = m1 + 1.1503494 * sigma

    lo = jnp.zeros((1, TB), dtype=jnp.uint32)
    hi = jnp.full((1, TB), jnp.uint32(0xFFFFFFFF))
    thr = jnp.zeros((1, TB), dtype=jnp.uint32)
    done = jnp.zeros((1, TB), dtype=jnp.int32)

    def absorb(midu, cnt, lo, hi, thr, done):
        ge = cnt >= K
        nlo = jnp.where(ge, midu, lo)
        nhi = jnp.where(ge, hi, midu)
        closed = nlo + jnp.uint32(1) >= nhi
        hit = cnt == K
        nthr = jnp.where((done == 0) & (hit | closed),
                         jnp.where(hit, midu, nlo), thr)
        ndone = jnp.where(hit | closed, jnp.int32(1), done)
        return nlo, nhi, nthr, ndone

    c0 = count_ge(enc(t0))
    lo, hi, thr, done = absorb(enc(t0), c0, lo, hi, thr, done)
    scale = sigma * (1.0 / (I * 0.20594))
    t1 = t0 + (c0 - K).astype(jnp.float32) * scale
    margin = 100.0 * scale
    tb = t1 - margin
    cb = count_ge(enc(tb))
    lo, hi, thr, done = absorb(enc(tb), cb, lo, hi, thr, done)
    ta = t1 + margin
    ca = count_ge(enc(ta))
    lo, hi, thr, done = absorb(enc(ta), ca, lo, hi, thr, done)

    def cond(state):
        i, lo, hi, thr, done = state
        return jnp.logical_and(i < 34, jnp.any(done == 0))

    def body(state):
        i, lo, hi, thr, done = state
        midu = lo + ((hi - lo) >> 1)
        cnt = count_ge(midu)
        lo, hi, thr, done = absorb(midu, cnt, lo, hi, thr, done)
        return (i + 1, lo, hi, thr, done)

    _, lo, hi, thr, done = lax.while_loop(
        cond, body, (jnp.int32(0), lo, hi, thr, done))

    mask = ukey >= thr
    act = gate * jax.nn.sigmoid(gate) * up
    masked = jnp.where(mask, act, 0.0).astype(jnp.bfloat16)
    o_ref[...] = lax.dot_general(masked, dw_ref[...],
                                 (((0,), (1,)), ((), ())),
                                 preferred_element_type=jnp.float32)


@jax.jit
def kernel(x, gate_w, up_w, down_w):
    b, s, h = x.shape
    T = b * s
    x2 = x.reshape(T, h)
    up_w = up_w.astype(jnp.bfloat16)
    down_w = down_w.astype(jnp.bfloat16)
    out = pl.pallas_call(
        _moc_body,
        grid=(T // TB,),
        in_specs=[
            pl.BlockSpec((TB, H), lambda i: (i, 0)),
            pl.BlockSpec((I, H), lambda i: (0, 0)),
            pl.BlockSpec((I, H), lambda i: (0, 0)),
            pl.BlockSpec((H, I), lambda i: (0, 0)),
        ],
        out_specs=pl.BlockSpec((TB, H), lambda i: (i, 0)),
        out_shape=jax.ShapeDtypeStruct((T, H), jnp.float32),
    )(x2, gate_w, up_w, down_w)
    return out.reshape(b, s, h)


# speedup vs baseline: 1.3354x; 1.0683x over previous
import functools
import jax
import jax.numpy as jnp
from jax import lax
from jax.experimental import pallas as pl
from jax.experimental.pallas import tpu as pltpu

B, S, H, I, K = 4, 2048, 768, 3072, 384
TB = 256


def _moc_body(x_ref, gw_ref, uw_ref, dw_ref, o_ref):
    xb = x_ref[...]
    gate = lax.dot_general(gw_ref[...], xb,
                           (((1,), (1,)), ((), ())),
                           preferred_element_type=jnp.float32)
    up = lax.dot_general(uw_ref[...], xb.astype(jnp.bfloat16),
                         (((1,), (1,)), ((), ())),
                         preferred_element_type=jnp.float32)

    def enc(f):
        b = lax.bitcast_convert_type(f, jnp.uint32)
        return jnp.where(b >> 31 == 1, ~b, b | jnp.uint32(0x80000000))

    ukey = enc(gate)

    def count_ge(cand):
        return jnp.sum((ukey >= cand).astype(jnp.int32), axis=0, keepdims=True)

    m1 = jnp.sum(gate, axis=0, keepdims=True) * (1.0 / I)
    m2 = jnp.sum(gate * gate, axis=0, keepdims=True) * (1.0 / I)
    sigma = jnp.sqrt(jnp.maximum(m2 - m1 * m1, 1e-30))
    t0 = m1 + 1.1503494 * sigma

    lo = jnp.zeros((1, TB), dtype=jnp.uint32)
    hi = jnp.full((1, TB), jnp.uint32(0xFFFFFFFF))
    thr = jnp.zeros((1, TB), dtype=jnp.uint32)
    done = jnp.zeros((1, TB), dtype=jnp.int32)

    def absorb(midu, cnt, lo, hi, thr, done):
        ge = cnt >= K
        nlo = jnp.where(ge, midu, lo)
        nhi = jnp.where(ge, hi, midu)
        closed = nlo + jnp.uint32(1) >= nhi
        hit = cnt == K
        nthr = jnp.where((done == 0) & (hit | closed),
                         jnp.where(hit, midu, nlo), thr)
        ndone = jnp.where(hit | closed, jnp.int32(1), done)
        return nlo, nhi, nthr, ndone

    c0 = count_ge(enc(t0))
    lo, hi, thr, done = absorb(enc(t0), c0, lo, hi, thr, done)
    scale = sigma * (1.0 / (I * 0.20594))
    t1 = t0 + (c0 - K).astype(jnp.float32) * scale
    margin = 64.0 * scale
    tb = t1 - margin
    cb = count_ge(enc(tb))
    lo, hi, thr, done = absorb(enc(tb), cb, lo, hi, thr, done)
    ta = t1 + margin
    ca = count_ge(enc(ta))
    lo, hi, thr, done = absorb(enc(ta), ca, lo, hi, thr, done)

    def cond(state):
        i, lo, hi, thr, done = state
        return jnp.logical_and(i < 34, jnp.any(done == 0))

    def body(state):
        i, lo, hi, thr, done = state
        for _ in range(4):
            midu = lo + ((hi - lo) >> 1)
            cnt = count_ge(midu)
            lo, hi, thr, done = absorb(midu, cnt, lo, hi, thr, done)
        return (i + 1, lo, hi, thr, done)

    _, lo, hi, thr, done = lax.while_loop(
        cond, body, (jnp.int32(0), lo, hi, thr, done))

    thr = jnp.where(done == 1, thr, lo)

    mask = ukey >= thr
    act = gate * jax.nn.sigmoid(gate) * up
    masked = jnp.where(mask, act, 0.0).astype(jnp.bfloat16)
    o_ref[...] = lax.dot_general(masked, dw_ref[...],
                                 (((0,), (1,)), ((), ())),
                                 preferred_element_type=jnp.float32)


@jax.jit
def kernel(x, gate_w, up_w, down_w):
    b, s, h = x.shape
    T = b * s
    x2 = x.reshape(T, h)
    up_w = up_w.astype(jnp.bfloat16)
    down_w = down_w.astype(jnp.bfloat16)
    out = pl.pallas_call(
        _moc_body,
        grid=(T // TB,),
        in_specs=[
            pl.BlockSpec((TB, H), lambda i: (i, 0)),
            pl.BlockSpec((I, H), lambda i: (0, 0)),
            pl.BlockSpec((I, H), lambda i: (0, 0)),
            pl.BlockSpec((H, I), lambda i: (0, 0)),
        ],
        out_specs=pl.BlockSpec((TB, H), lambda i: (i, 0)),
        out_shape=jax.ShapeDtypeStruct((T, H), jnp.float32),
    )(x2, gate_w, up_w, down_w)
    return out.reshape(b, s, h)
